# baseline (speedup 1.0000x reference)
import jax
import jax.numpy as jnp
from jax.experimental import pallas as pl
from jax.experimental.pallas import tpu as pltpu

N = 2048
EMB = 64
NF = 16
GS = 128
T = 4
NV = 512
NQ = 256
PREP_BLK = 256


def _prep_kernel(e_ref, nf_ref, w1t_ref, w3t_ref, w4_ref, out_ref):
    e_sum = jnp.sum(e_ref[...], axis=1, keepdims=True)
    t3 = jax.nn.relu(e_sum * w4_ref[...])
    t3 = jnp.dot(t3, w3t_ref[...], preferred_element_type=jnp.float32)
    t1 = jnp.dot(nf_ref[...], w1t_ref[...], preferred_element_type=jnp.float32)
    out_ref[...] = t1 + t3


def _main_kernel(a_ref, t13_ref, emb0_ref, idx_ref,
                 w2t_ref, w51_ref, w52t_ref, w6t_ref, w7t_ref, w8t_ref, w9t_ref,
                 emb_out_ref, q_out_ref, delta_out_ref):
    a = a_ref[...]
    deg = jnp.sum(a, axis=1, keepdims=True)
    inv_deg = 1.0 / deg
    t13 = t13_ref[...]
    emb = emb0_ref[...]
    w2t = w2t_ref[...]
    w8t = w8t_ref[...]
    w9t = w9t_ref[...]

    row = jax.lax.broadcasted_iota(jnp.int32, (N, 1), 0)
    hi_mask = (row >= NV).astype(jnp.float32)
    lo_mask = (row < NV).astype(jnp.float32)

    def get_state(e):
        s = jnp.sum(e * lo_mask, axis=0, keepdims=True)
        return jnp.dot(s, w9t, preferred_element_type=jnp.float32)

    state = get_state(emb)
    prev = emb
    for _ in range(T):
        prev = emb
        z = jnp.dot(a, emb, preferred_element_type=jnp.float32)
        t2 = jnp.dot(z, w2t, preferred_element_type=jnp.float32) * inv_deg
        sterm = jnp.dot(state, w8t, preferred_element_type=jnp.float32)
        t2 = t2 + hi_mask * sterm
        emb = jax.nn.relu(t13 + t2)
        state = get_state(emb)

    emb_out_ref[...] = emb
    delta_out_ref[...] = prev - emb

    idx = idx_ref[...]
    col = jax.lax.broadcasted_iota(jnp.int32, (NQ, N), 1)
    onehot = (col == idx).astype(jnp.float32)
    gathered = jnp.dot(onehot, emb, preferred_element_type=jnp.float32)
    b_ = jax.nn.relu(jnp.dot(gathered, w7t_ref[...],
                             preferred_element_type=jnp.float32))
    a_ = jax.nn.relu(jnp.dot(state, w6t_ref[...],
                             preferred_element_type=jnp.float32))
    aq = jnp.sum(a_ * w51_ref[...])
    q = jnp.dot(b_, w52t_ref[...], preferred_element_type=jnp.float32) + aq
    q_out_ref[...] = q.T


def kernel(node_feature_matrix, adjacency_matrix, edge_feature_matrix, current_embedding,
           unassigned_decision_variable_indices, assigned_variable_indices,
           W1, W2, W3, W4, W5_1, W5_2, W6, W7, W8, W9):
    e2d = edge_feature_matrix.reshape(N, N)
    idx2d = unassigned_decision_variable_indices.reshape(NQ, 1)

    term13 = pl.pallas_call(
        _prep_kernel,
        grid=(N // PREP_BLK,),
        in_specs=[
            pl.BlockSpec((PREP_BLK, N), lambda i: (i, 0)),
            pl.BlockSpec((PREP_BLK, NF), lambda i: (i, 0)),
            pl.BlockSpec((NF, EMB), lambda i: (0, 0)),
            pl.BlockSpec((EMB, EMB), lambda i: (0, 0)),
            pl.BlockSpec((1, EMB), lambda i: (0, 0)),
        ],
        out_specs=pl.BlockSpec((PREP_BLK, EMB), lambda i: (i, 0)),
        out_shape=jax.ShapeDtypeStruct((N, EMB), jnp.float32),
    )(e2d, node_feature_matrix, W1.T, W3.T, W4.T)

    emb_out, q_t, delta = pl.pallas_call(
        _main_kernel,
        out_shape=(
            jax.ShapeDtypeStruct((N, EMB), jnp.float32),
            jax.ShapeDtypeStruct((2, NQ), jnp.float32),
            jax.ShapeDtypeStruct((N, EMB), jnp.float32),
        ),
        compiler_params=pltpu.CompilerParams(vmem_limit_bytes=50 * 1024 * 1024),
    )(adjacency_matrix, term13, current_embedding, idx2d,
      W2.T, W5_1, W5_2.T, W6.T, W7.T, W8.T, W9.T)

    return (emb_out, q_t, delta)


# speedup vs baseline: 1.3853x; 1.3853x over previous
import jax
import jax.numpy as jnp
from jax.experimental import pallas as pl
from jax.experimental.pallas import tpu as pltpu

N = 2048
EMB = 64
NF = 16
GS = 128
T = 4
NV = 512
NQ = 256
RB = 128
NBLK = N // RB
CHUNKS = N // 128


def _bdot(x, y):
    return jax.lax.dot_general(
        x.astype(jnp.bfloat16), y.astype(jnp.bfloat16),
        (((x.ndim - 1,), (0,)), ((), ())),
        preferred_element_type=jnp.float32)


def _fused_kernel(a_ref, e_ref, nf_ref, emb0_ref, idx_ref,
                  w1t_ref, w2t_ref, w3t_ref, w4_ref, w51_ref, w52t_ref,
                  w6t_ref, w7t_ref, w8t_ref, w9t_ref,
                  emb_out_ref, q_out_ref, delta_out_ref,
                  a_bf_s, t13_s, z1_s, deg_s):
    i = pl.program_id(0)
    rows = pl.ds(i * RB, RB)

    a_blk = a_ref[...]
    a_bf_blk = a_blk.astype(jnp.bfloat16)
    a_bf_s[rows, :] = a_bf_blk
    deg_s[rows, :] = jnp.sum(a_blk, axis=1, keepdims=True)

    emb0_bf = emb0_ref[...].astype(jnp.bfloat16)
    z1_s[rows, :] = jax.lax.dot_general(
        a_bf_blk, emb0_bf, (((1,), (0,)), ((), ())),
        preferred_element_type=jnp.float32)

    x = e_ref[...]
    ones_col = jnp.ones((128, 1), jnp.float32)
    s_chunk = jnp.dot(x, ones_col, preferred_element_type=jnp.float32)
    r_i = jax.lax.broadcasted_iota(jnp.int32, (RB, RB * CHUNKS), 0)
    c_i = jax.lax.broadcasted_iota(jnp.int32, (RB, RB * CHUNKS), 1)
    grp = jnp.where(c_i // CHUNKS == r_i, 1.0, 0.0)
    e_sum = jnp.dot(grp, s_chunk, preferred_element_type=jnp.float32)
    e_bf = e_sum.astype(jnp.bfloat16).astype(jnp.float32)
    w4_bf = w4_ref[...].astype(jnp.bfloat16).astype(jnp.float32)
    t3 = jax.nn.relu(e_bf * w4_bf)
    t13_s[rows, :] = _bdot(t3, w3t_ref[...]) + _bdot(nf_ref[...], w1t_ref[...])

    @pl.when(i == NBLK - 1)
    def _():
        deg = deg_s[...]
        t13 = t13_s[...]
        w2t = w2t_ref[...]
        w8t = w8t_ref[...]
        w9t = w9t_ref[...]
        row = jax.lax.broadcasted_iota(jnp.int32, (N, 1), 0)
        hi_mask = (row >= NV).astype(jnp.float32)
        ones_row = jnp.ones((1, NV), jnp.float32)

        def get_state(e):
            s = jnp.dot(ones_row, e[:NV], preferred_element_type=jnp.float32)
            return _bdot(s, w9t)

        emb0 = emb0_ref[...]
        state = get_state(emb0)
        a_bf = a_bf_s[...]
        prev = emb0
        emb = emb0
        for t in range(T):
            prev = emb
            if t == 0:
                z = z1_s[...]
            else:
                z = jax.lax.dot_general(
                    a_bf, emb.astype(jnp.bfloat16), (((1,), (0,)), ((), ())),
                    preferred_element_type=jnp.float32)
            t2 = _bdot(z, w2t) / deg
            sterm = _bdot(state, w8t)
            t2 = t2 + hi_mask * sterm
            emb = jax.nn.relu(t13 + t2)
            state = get_state(emb)

        emb_out_ref[...] = emb
        delta_out_ref[...] = prev - emb

        idx = idx_ref[...].reshape(1, NQ)
        row_i = jax.lax.broadcasted_iota(jnp.int32, (N, NQ), 0)
        onehot_t = (row_i == idx).astype(jnp.float32)
        gathered = jax.lax.dot_general(
            onehot_t, emb, (((0,), (0,)), ((), ())),
            preferred_element_type=jnp.float32)
        b_ = jax.nn.relu(_bdot(gathered, w7t_ref[...]))
        a_ = jax.nn.relu(_bdot(state, w6t_ref[...]))
        a_bfv = a_.astype(jnp.bfloat16).astype(jnp.float32)
        w51_bf = w51_ref[...].astype(jnp.bfloat16).astype(jnp.float32)
        aq = jnp.sum(a_bfv * w51_bf)
        q = _bdot(b_, w52t_ref[...]) + aq
        q_out_ref[...] = q.T


def kernel(node_feature_matrix, adjacency_matrix, edge_feature_matrix, current_embedding,
           unassigned_decision_variable_indices, assigned_variable_indices,
           W1, W2, W3, W4, W5_1, W5_2, W6, W7, W8, W9):
    e128 = edge_feature_matrix.reshape(N * CHUNKS, 128)

    emb_out, q_t, delta = pl.pallas_call(
        _fused_kernel,
        grid=(NBLK,),
        in_specs=[
            pl.BlockSpec((RB, N), lambda i: (i, 0)),
            pl.BlockSpec((RB * CHUNKS, 128), lambda i: (i, 0)),
            pl.BlockSpec((RB, NF), lambda i: (i, 0)),
            pl.BlockSpec((N, EMB), lambda i: (0, 0)),
            pl.BlockSpec((NQ,), lambda i: (0,)),
            pl.BlockSpec((NF, EMB), lambda i: (0, 0)),
            pl.BlockSpec((EMB, EMB), lambda i: (0, 0)),
            pl.BlockSpec((EMB, EMB), lambda i: (0, 0)),
            pl.BlockSpec((1, EMB), lambda i: (0, 0)),
            pl.BlockSpec((1, EMB), lambda i: (0, 0)),
            pl.BlockSpec((EMB, 2), lambda i: (0, 0)),
            pl.BlockSpec((GS, EMB), lambda i: (0, 0)),
            pl.BlockSpec((EMB, EMB), lambda i: (0, 0)),
            pl.BlockSpec((GS, EMB), lambda i: (0, 0)),
            pl.BlockSpec((EMB, GS), lambda i: (0, 0)),
        ],
        out_specs=(
            pl.BlockSpec((N, EMB), lambda i: (0, 0)),
            pl.BlockSpec((2, NQ), lambda i: (0, 0)),
            pl.BlockSpec((N, EMB), lambda i: (0, 0)),
        ),
        out_shape=(
            jax.ShapeDtypeStruct((N, EMB), jnp.float32),
            jax.ShapeDtypeStruct((2, NQ), jnp.float32),
            jax.ShapeDtypeStruct((N, EMB), jnp.float32),
        ),
        scratch_shapes=[
            pltpu.VMEM((N, N), jnp.bfloat16),
            pltpu.VMEM((N, EMB), jnp.float32),
            pltpu.VMEM((N, EMB), jnp.float32),
            pltpu.VMEM((N, 1), jnp.float32),
        ],
        compiler_params=pltpu.CompilerParams(vmem_limit_bytes=50 * 1024 * 1024),
    )(adjacency_matrix, e128, node_feature_matrix, current_embedding,
      unassigned_decision_variable_indices,
      W1.T, W2.T, W3.T, W4.T, W5_1, W5_2.T, W6.T, W7.T, W8.T, W9.T)

    return (emb_out, q_t, delta)


# speedup vs baseline: 1.4857x; 1.0725x over previous
import jax
import jax.numpy as jnp
from jax.experimental import pallas as pl
from jax.experimental.pallas import tpu as pltpu

N = 2048
EMB = 64
NF = 16
GS = 128
T = 4
NV = 512
NQ = 256
RB = 128
NBLK = N // RB
CHUNKS = N // 128


def _fused_kernel(a_ref, e_ref, nf_ref, emb0_ref, idx_ref,
                  w1t_ref, w2t_ref, w3t_ref, w4_ref, w51_ref, w52t_ref,
                  w6t_ref, w7t_ref, w8t_ref, w9t_ref,
                  emb_out_ref, q_out_ref, delta_out_ref,
                  a_s, t1_s, t3_s, z1_s, deg_s):
    i = pl.program_id(0)
    rows = pl.ds(i * RB, RB)

    a_blk = a_ref[...]
    a_s[rows, :] = a_blk
    deg_s[rows, :] = jnp.sum(a_blk, axis=1, keepdims=True)

    z1_s[rows, :] = jnp.dot(a_blk, emb0_ref[...],
                            preferred_element_type=jnp.float32)

    x = e_ref[...].reshape(RB, CHUNKS, 128)
    e_sum = jnp.sum(jnp.sum(x, axis=1), axis=1, keepdims=True)
    t3 = jax.nn.relu(e_sum * w4_ref[...])
    t3_s[rows, :] = jnp.dot(t3, w3t_ref[...], preferred_element_type=jnp.float32)
    t1_s[rows, :] = jnp.dot(nf_ref[...], w1t_ref[...], preferred_element_type=jnp.float32)

    @pl.when(i == NBLK - 1)
    def _():
        deg = deg_s[...]
        t1 = t1_s[...]
        t3f = t3_s[...]
        w2t = w2t_ref[...]
        w8t = w8t_ref[...]
        w9t = w9t_ref[...]
        row = jax.lax.broadcasted_iota(jnp.int32, (N, 1), 0)
        hi_mask = (row >= NV).astype(jnp.float32)

        def get_state(e):
            s = jnp.sum(e[:NV], axis=0, keepdims=True)
            return jnp.dot(s, w9t, preferred_element_type=jnp.float32)

        emb0 = emb0_ref[...]
        state = get_state(emb0)
        a = a_s[...]
        prev = emb0
        emb = emb0
        for t in range(T):
            prev = emb
            if t == 0:
                z = z1_s[...]
            else:
                z = jnp.dot(a, emb, preferred_element_type=jnp.float32)
            t2 = jnp.dot(z, w2t, preferred_element_type=jnp.float32) / deg
            sterm = jnp.dot(state, w8t, preferred_element_type=jnp.float32)
            t2 = t2 + hi_mask * sterm
            emb = jax.nn.relu((t1 + t2) + t3f)
            state = get_state(emb)

        emb_out_ref[...] = emb
        delta_out_ref[...] = prev - emb

        idx = idx_ref[...].reshape(1, NQ)
        row_i = jax.lax.broadcasted_iota(jnp.int32, (N, NQ), 0)
        onehot_t = (row_i == idx).astype(jnp.float32)
        gathered = jax.lax.dot_general(
            onehot_t, emb, (((0,), (0,)), ((), ())),
            preferred_element_type=jnp.float32)
        b_ = jax.nn.relu(jnp.dot(gathered, w7t_ref[...],
                                 preferred_element_type=jnp.float32))
        a_ = jax.nn.relu(jnp.dot(state, w6t_ref[...],
                                 preferred_element_type=jnp.float32))
        aq = jnp.sum(a_ * w51_ref[...])
        q = jnp.dot(b_, w52t_ref[...], preferred_element_type=jnp.float32) + aq
        q_out_ref[...] = q.T


def kernel(node_feature_matrix, adjacency_matrix, edge_feature_matrix, current_embedding,
           unassigned_decision_variable_indices, assigned_variable_indices,
           W1, W2, W3, W4, W5_1, W5_2, W6, W7, W8, W9):
    e128 = edge_feature_matrix.reshape(N * CHUNKS, 128)

    emb_out, q_t, delta = pl.pallas_call(
        _fused_kernel,
        grid=(NBLK,),
        in_specs=[
            pl.BlockSpec((RB, N), lambda i: (i, 0)),
            pl.BlockSpec((RB * CHUNKS, 128), lambda i: (i, 0)),
            pl.BlockSpec((RB, NF), lambda i: (i, 0)),
            pl.BlockSpec((N, EMB), lambda i: (0, 0)),
            pl.BlockSpec((NQ,), lambda i: (0,)),
            pl.BlockSpec((NF, EMB), lambda i: (0, 0)),
            pl.BlockSpec((EMB, EMB), lambda i: (0, 0)),
            pl.BlockSpec((EMB, EMB), lambda i: (0, 0)),
            pl.BlockSpec((1, EMB), lambda i: (0, 0)),
            pl.BlockSpec((1, EMB), lambda i: (0, 0)),
            pl.BlockSpec((EMB, 2), lambda i: (0, 0)),
            pl.BlockSpec((GS, EMB), lambda i: (0, 0)),
            pl.BlockSpec((EMB, EMB), lambda i: (0, 0)),
            pl.BlockSpec((GS, EMB), lambda i: (0, 0)),
            pl.BlockSpec((EMB, GS), lambda i: (0, 0)),
        ],
        out_specs=(
            pl.BlockSpec((N, EMB), lambda i: (0, 0)),
            pl.BlockSpec((2, NQ), lambda i: (0, 0)),
            pl.BlockSpec((N, EMB), lambda i: (0, 0)),
        ),
        out_shape=(
            jax.ShapeDtypeStruct((N, EMB), jnp.float32),
            jax.ShapeDtypeStruct((2, NQ), jnp.float32),
            jax.ShapeDtypeStruct((N, EMB), jnp.float32),
        ),
        scratch_shapes=[
            pltpu.VMEM((N, N), jnp.float32),
            pltpu.VMEM((N, EMB), jnp.float32),
            pltpu.VMEM((N, EMB), jnp.float32),
            pltpu.VMEM((N, EMB), jnp.float32),
            pltpu.VMEM((N, 1), jnp.float32),
        ],
        compiler_params=pltpu.CompilerParams(vmem_limit_bytes=50 * 1024 * 1024),
    )(adjacency_matrix, e128, node_feature_matrix, current_embedding,
      unassigned_decision_variable_indices,
      W1.T, W2.T, W3.T, W4.T, W5_1, W5_2.T, W6.T, W7.T, W8.T, W9.T)

    return (emb_out, q_t, delta)
